# Initial kernel scaffold; baseline (speedup 1.0000x reference)
import functools

import jax
import jax.numpy as jnp
from jax import lax
from jax.experimental import pallas as pl
from jax.experimental.pallas import tpu as pltpu
from jax.experimental.pallas import tpu_sc as plsc

B, U, L, D, P = 1024, 20, 50, 64, 200

NC, NS = 2, 16
NW = NC * NS
BPW = B // NW
LP = 64
PAIR = 2
NSTEP = BPW // PAIR


def _sc_gather(gidx, obs_rows):
  mesh = plsc.VectorSubcoreMesh(core_axis_name="c", subcore_axis_name="s")

  @functools.partial(
      pl.kernel,
      out_type=jax.ShapeDtypeStruct((B, L, D), jnp.float32),
      mesh=mesh,
      scratch_types=[
          pltpu.VMEM((NSTEP, PAIR * LP), jnp.int32),
          pltpu.VMEM((PAIR * LP, D), jnp.float32),
          pltpu.SemaphoreType.DMA,
      ],
  )
  def k(gidx_hbm, obs_hbm, out_hbm, idx_v, rows_v, sem):
    wid = lax.axis_index("s") * NC + lax.axis_index("c")
    base_b = wid * BPW
    pltpu.sync_copy(gidx_hbm.at[wid], idx_v)

    def step(p, carry):
      b0 = base_b + p * PAIR
      pltpu.async_copy(obs_hbm.at[idx_v.at[p]], rows_v, sem).wait()
      pltpu.sync_copy(rows_v.at[pl.ds(0, L)], out_hbm.at[b0])
      pltpu.sync_copy(rows_v.at[pl.ds(LP, L)], out_hbm.at[b0 + 1])
      return carry

    lax.fori_loop(0, NSTEP, step, 0)

  return k(gidx, obs_rows)


def _tc_matmul(utterance, gathered):
  BB = 16

  def body(utt_ref, g_ref, out_ref):
    for j in range(BB):
      out_ref[j] = lax.dot_general(
          utt_ref[j], g_ref[j],
          (((1,), (1,)), ((), ())),
          preferred_element_type=jnp.float32)

  return pl.pallas_call(
      body,
      grid=(B // BB,),
      in_specs=[
          pl.BlockSpec((BB, U, D), lambda i: (i, 0, 0)),
          pl.BlockSpec((BB, L, D), lambda i: (i, 0, 0)),
      ],
      out_specs=pl.BlockSpec((BB, U, L), lambda i: (i, 0, 0)),
      out_shape=jax.ShapeDtypeStruct((B, U, L), jnp.float32),
  )(utterance, gathered)


def kernel(utterance, world, observation):
  world = world.astype(jnp.int32)
  gidx = world + (jnp.arange(B, dtype=jnp.int32) * P)[:, None]
  pad = jnp.broadcast_to((jnp.arange(B, dtype=jnp.int32) * P)[:, None],
                         (B, LP - L))
  gidx_pad = jnp.concatenate([gidx, pad], axis=1)
  gidx_pad = gidx_pad.reshape(NW, NSTEP, PAIR * LP)
  gathered = _sc_gather(gidx_pad, observation.reshape(B * P, D))
  return _tc_matmul(utterance, gathered)


# speedup vs baseline: 2.2565x; 2.2565x over previous
import functools

import jax
import jax.numpy as jnp
from jax import lax
from jax.experimental import pallas as pl
from jax.experimental.pallas import tpu as pltpu
from jax.experimental.pallas import tpu_sc as plsc

B, U, L, D, P = 1024, 20, 50, 64, 200

NC, NS = 2, 16
NW = NC * NS
BPW = B // NW
LP = 64
PAIR = 2
NSTEP = BPW // PAIR


def _sc_gather(gidx, obs_rows):
  mesh = plsc.VectorSubcoreMesh(core_axis_name="c", subcore_axis_name="s")

  @functools.partial(
      pl.kernel,
      out_type=jax.ShapeDtypeStruct((B, L, D), jnp.float32),
      mesh=mesh,
      scratch_types=[
          pltpu.VMEM((NSTEP, PAIR * LP), jnp.int32),
          pltpu.VMEM((PAIR * LP, D), jnp.float32),
          pltpu.SemaphoreType.DMA,
      ],
      compiler_params=pltpu.CompilerParams(use_tc_tiling_on_sc=False),
  )
  def k(gidx_hbm, obs_hbm, out_hbm, idx_v, rows_v, sem):
    wid = lax.axis_index("s") * NC + lax.axis_index("c")
    base_b = wid * BPW
    pltpu.sync_copy(gidx_hbm.at[wid], idx_v)

    def step(p, carry):
      b0 = base_b + p * PAIR
      pltpu.async_copy(obs_hbm.at[idx_v.at[p]], rows_v, sem).wait()
      pltpu.sync_copy(rows_v.at[pl.ds(0, L)], out_hbm.at[b0])
      pltpu.sync_copy(rows_v.at[pl.ds(LP, L)], out_hbm.at[b0 + 1])
      return carry

    lax.fori_loop(0, NSTEP, step, 0)

  return k(gidx, obs_rows)


def _tc_matmul(utterance, gathered):
  BB = 16

  def body(utt_ref, g_ref, out_ref):
    for j in range(BB):
      out_ref[j] = lax.dot_general(
          utt_ref[j], g_ref[j],
          (((1,), (1,)), ((), ())),
          preferred_element_type=jnp.float32)

  return pl.pallas_call(
      body,
      grid=(B // BB,),
      in_specs=[
          pl.BlockSpec((BB, U, D), lambda i: (i, 0, 0)),
          pl.BlockSpec((BB, L, D), lambda i: (i, 0, 0)),
      ],
      out_specs=pl.BlockSpec((BB, U, L), lambda i: (i, 0, 0)),
      out_shape=jax.ShapeDtypeStruct((B, U, L), jnp.float32),
  )(utterance, gathered)


def kernel(utterance, world, observation):
  world = world.astype(jnp.int32)
  gidx = world + (jnp.arange(B, dtype=jnp.int32) * P)[:, None]
  pad = jnp.broadcast_to((jnp.arange(B, dtype=jnp.int32) * P)[:, None],
                         (B, LP - L))
  gidx_pad = jnp.concatenate([gidx, pad], axis=1)
  gidx_pad = gidx_pad.reshape(NW, NSTEP, PAIR * LP)
  gathered = _sc_gather(gidx_pad, observation.reshape(B * P, D))
  return _tc_matmul(utterance, gathered)


# speedup vs baseline: 2.5436x; 1.1272x over previous
import functools

import jax
import jax.numpy as jnp
from jax import lax
from jax.experimental import pallas as pl
from jax.experimental.pallas import tpu as pltpu
from jax.experimental.pallas import tpu_sc as plsc

B, U, L, D, P = 1024, 20, 50, 64, 200

NC, NS = 2, 16
NW = NC * NS
BPW = B // NW
PAIR = 2
SPB = PAIR * L
NSTEP = BPW // PAIR


def _sc_gather(gidx, obs_rows):
  mesh = plsc.VectorSubcoreMesh(core_axis_name="c", subcore_axis_name="s")

  @functools.partial(
      pl.kernel,
      out_type=jax.ShapeDtypeStruct((B * L, D), jnp.float32),
      mesh=mesh,
      scratch_types=[
          pltpu.VMEM((NSTEP, SPB), jnp.int32),
          pltpu.VMEM((BPW * L, D), jnp.float32),
          pltpu.SemaphoreType.DMA,
      ],
      compiler_params=pltpu.CompilerParams(use_tc_tiling_on_sc=False),
  )
  def k(gidx_hbm, obs_hbm, out_hbm, idx_v, rows_v, sem):
    wid = lax.axis_index("s") * NC + lax.axis_index("c")
    pltpu.sync_copy(gidx_hbm.at[wid], idx_v)
    copies = [
        pltpu.async_copy(obs_hbm.at[idx_v.at[p]],
                         rows_v.at[pl.ds(p * SPB, SPB)], sem)
        for p in range(NSTEP)
    ]
    for c in copies:
      c.wait()
    pltpu.sync_copy(rows_v, out_hbm.at[pl.ds(wid * BPW * L, BPW * L)])

  return k(gidx, obs_rows)


def _tc_matmul(utterance, gathered):
  BB = 16

  def body(utt_ref, g_ref, out_ref):
    for j in range(BB):
      out_ref[j] = lax.dot_general(
          utt_ref[j], g_ref[j],
          (((1,), (1,)), ((), ())),
          preferred_element_type=jnp.float32)

  return pl.pallas_call(
      body,
      grid=(B // BB,),
      in_specs=[
          pl.BlockSpec((BB, U, D), lambda i: (i, 0, 0)),
          pl.BlockSpec((BB, L, D), lambda i: (i, 0, 0)),
      ],
      out_specs=pl.BlockSpec((BB, U, L), lambda i: (i, 0, 0)),
      out_shape=jax.ShapeDtypeStruct((B, U, L), jnp.float32),
  )(utterance, gathered)


def kernel(utterance, world, observation):
  world = world.astype(jnp.int32)
  gidx = world + (jnp.arange(B, dtype=jnp.int32) * P)[:, None]
  gidx = gidx.reshape(NW, NSTEP, SPB)
  gathered = _sc_gather(gidx, observation.reshape(B * P, D))
  return _tc_matmul(utterance, gathered.reshape(B, L, D))


# speedup vs baseline: 3.6858x; 1.4490x over previous
import functools

import jax
import jax.numpy as jnp
from jax import lax
from jax.experimental import pallas as pl
from jax.experimental.pallas import tpu as pltpu
from jax.experimental.pallas import tpu_sc as plsc

B, U, L, D, P = 1024, 20, 50, 64, 200

NC, NS = 2, 16
NW = NC * NS
BPW = B // NW
SLOT = 56
PHB = 16
ROWS128 = B * P * D // 128


def _sc_gather(jidx, obs):
  mesh = plsc.VectorSubcoreMesh(core_axis_name="c", subcore_axis_name="s")

  @functools.partial(
      pl.kernel,
      out_type=jax.ShapeDtypeStruct((B * SLOT, 128), jnp.float32),
      mesh=mesh,
      scratch_types=[
          pltpu.VMEM((BPW, L), jnp.int32),
          pltpu.VMEM((PHB * SLOT, 128), jnp.float32),
          pltpu.SemaphoreType.DMA,
      ],
      compiler_params=pltpu.CompilerParams(use_tc_tiling_on_sc=True),
  )
  def k(jidx_hbm, obs_hbm, out_hbm, idx_v, rows_v, sem):
    wid = lax.axis_index("s") * NC + lax.axis_index("c")
    pltpu.sync_copy(jidx_hbm.at[wid], idx_v)
    tbl = obs_hbm.reshape(ROWS128, 128)
    for ph in range(BPW // PHB):
      copies = [
          pltpu.async_copy(tbl.at[idx_v.at[ph * PHB + p]],
                           rows_v.at[pl.ds(p * SLOT, L)], sem)
          for p in range(PHB)
      ]
      for c in copies:
        c.wait()
      pltpu.sync_copy(
          rows_v,
          out_hbm.at[pl.ds((wid * BPW + ph * PHB) * SLOT, PHB * SLOT)])

  return k(jidx, obs)


def _tc_matmul(utterance, gp, hsel):
  BB = 16

  def body(utt_ref, gp_ref, h_ref, out_ref):
    for j in range(BB):
      rows = gp_ref[pl.ds(j * SLOT, L), :]
      a = utt_ref[j]
      oe = lax.dot_general(a, rows[:, :D], (((1,), (1,)), ((), ())),
                           preferred_element_type=jnp.float32)
      oo = lax.dot_general(a, rows[:, D:], (((1,), (1,)), ((), ())),
                           preferred_element_type=jnp.float32)
      m = h_ref[j]
      out_ref[j] = oe + m[None, :] * (oo - oe)

  return pl.pallas_call(
      body,
      grid=(B // BB,),
      in_specs=[
          pl.BlockSpec((BB, U, D), lambda i: (i, 0, 0)),
          pl.BlockSpec((BB * SLOT, 128), lambda i: (i, 0)),
          pl.BlockSpec((BB, L), lambda i: (i, 0)),
      ],
      out_specs=pl.BlockSpec((BB, U, L), lambda i: (i, 0, 0)),
      out_shape=jax.ShapeDtypeStruct((B, U, L), jnp.float32),
  )(utterance, gp, hsel)


def kernel(utterance, world, observation):
  world = world.astype(jnp.int32)
  b = jnp.arange(B, dtype=jnp.int32)[:, None]
  jidx = (b // 8) * 800 + (world // 2) * 8 + (b % 8)
  jidx = jidx.reshape(NW, BPW, L)
  hsel = (world % 2).astype(jnp.float32)
  obs4 = observation.reshape(128, 8, 100, 128).swapaxes(1, 2)
  gp = _sc_gather(jidx, obs4)
  return _tc_matmul(utterance, gp, hsel)


# speedup vs baseline: 4.7931x; 1.3004x over previous
import functools

import jax
import jax.numpy as jnp
from jax import lax
from jax.experimental import pallas as pl
from jax.experimental.pallas import tpu as pltpu
from jax.experimental.pallas import tpu_sc as plsc

B, U, L, D, P = 1024, 20, 50, 64, 200

NCHUNK = 2
BH = B // NCHUNK

NC, NS = 2, 16
NW = NC * NS
BPW = BH // NW
SLOT = 56
PHB = 8
ROWS128 = B * P * D // 128


def _sc_gather(jidx, obs):
  mesh = plsc.VectorSubcoreMesh(core_axis_name="c", subcore_axis_name="s")

  @functools.partial(
      pl.kernel,
      out_type=jax.ShapeDtypeStruct((BH * SLOT, 128), jnp.float32),
      mesh=mesh,
      scratch_types=[
          pltpu.VMEM((BPW, L), jnp.int32),
          pltpu.VMEM((PHB * SLOT, 128), jnp.float32),
          pltpu.SemaphoreType.DMA,
      ],
      compiler_params=pltpu.CompilerParams(use_tc_tiling_on_sc=True),
  )
  def k(jidx_hbm, obs_hbm, out_hbm, idx_v, rows_v, sem):
    wid = lax.axis_index("s") * NC + lax.axis_index("c")
    pltpu.sync_copy(jidx_hbm.at[wid], idx_v)
    tbl = obs_hbm.reshape(ROWS128, 128)
    for ph in range(BPW // PHB):
      copies = [
          pltpu.async_copy(tbl.at[idx_v.at[ph * PHB + p]],
                           rows_v.at[pl.ds(p * SLOT, L)], sem)
          for p in range(PHB)
      ]
      for c in copies:
        c.wait()
      pltpu.sync_copy(
          rows_v,
          out_hbm.at[pl.ds((wid * BPW + ph * PHB) * SLOT, PHB * SLOT)])

  return k(jidx, obs)


def _tc_matmul(utterance, gp, hsel):
  BB = 64

  def body(utt_ref, gp_ref, h_ref, out_ref):
    for j in range(BB):
      rows = gp_ref[pl.ds(j * SLOT, L), :]
      a = utt_ref[j]
      oe = lax.dot_general(a, rows[:, :D], (((1,), (1,)), ((), ())),
                           preferred_element_type=jnp.float32)
      oo = lax.dot_general(a, rows[:, D:], (((1,), (1,)), ((), ())),
                           preferred_element_type=jnp.float32)
      m = h_ref[j]
      out_ref[j] = oe + m[None, :] * (oo - oe)

  return pl.pallas_call(
      body,
      grid=(BH // BB,),
      in_specs=[
          pl.BlockSpec((BB, U, D), lambda i: (i, 0, 0)),
          pl.BlockSpec((BB * SLOT, 128), lambda i: (i, 0)),
          pl.BlockSpec((BB, L), lambda i: (i, 0)),
      ],
      out_specs=pl.BlockSpec((BB, U, L), lambda i: (i, 0, 0)),
      out_shape=jax.ShapeDtypeStruct((BH, U, L), jnp.float32),
  )(utterance, gp, hsel)


def kernel(utterance, world, observation):
  world = world.astype(jnp.int32)
  b = jnp.arange(B, dtype=jnp.int32)[:, None]
  jidx = (b // 8) * 800 + (world // 2) * 8 + (b % 8)
  hsel = (world % 2).astype(jnp.float32)
  obs4 = observation.reshape(128, 8, 100, 128).swapaxes(1, 2)
  outs = []
  for c in range(NCHUNK):
    sl = slice(c * BH, (c + 1) * BH)
    gp = _sc_gather(jidx[sl].reshape(NW, BPW, L), obs4)
    outs.append(_tc_matmul(utterance[sl], gp, hsel[sl]))
  return jnp.concatenate(outs, axis=0)
